# initial kernel scaffold (unmeasured)
import jax
import jax.numpy as jnp
from jax import lax
from jax.experimental import pallas as pl
from jax.experimental.pallas import tpu as pltpu

N_DEV = 8
SQ = 1024
SKV = 1024
H_PER = 8
DH = 128
D_MODEL = 1024
CHUNK = SQ // N_DEV
SCALE = 0.08838834764831843


def kernel(x, Wq, K_ext, V_ext, Wo):
    my = lax.axis_index("i")
    wq_my = lax.dynamic_slice(Wq, (0, my * H_PER * DH), (D_MODEL, H_PER * DH))
    wo_my = lax.dynamic_slice(Wo, (my * H_PER * DH, 0), (H_PER * DH, D_MODEL))
    x2 = x.reshape(SQ, D_MODEL)
    k3 = K_ext.reshape(SKV, H_PER, DH)
    v3 = V_ext.reshape(SKV, H_PER, DH)

    def body(x_ref, wq_ref, k_ref, v_ref, wo_ref, out_ref,
             acc_ref, ctx_ref, stage_ref, rs_recv_ref, ag_recv_ref,
             rs_send_sems, rs_recv_sems, ag_send_sems, ag_recv_sems):
        my_pos = lax.axis_index("i")
        right = lax.rem(my_pos + 1, N_DEV)

        xb = x_ref[:, :].astype(jnp.bfloat16)
        wqb = wq_ref[:, :].astype(jnp.bfloat16)
        q_all = jax.lax.dot(xb, wqb, preferred_element_type=jnp.float32)
        q_all = q_all.astype(jnp.bfloat16)

        qi = lax.broadcasted_iota(jnp.int32, (SQ, SKV), 0)
        ki = lax.broadcasted_iota(jnp.int32, (SQ, SKV), 1)
        mask = (jnp.abs(qi - ki) <= 128) | (ki < 32) | (qi < 32)

        for h in range(H_PER):
            q_h = q_all[:, h * DH:(h + 1) * DH]
            k_h = k_ref[:, h, :].astype(jnp.bfloat16)
            v_h = v_ref[:, h, :].astype(jnp.bfloat16)
            scores = lax.dot_general(
                q_h, k_h,
                dimension_numbers=(((1,), (1,)), ((), ())),
                preferred_element_type=jnp.float32,
            ) * SCALE
            scores = jnp.where(mask, scores, -1e9)
            m = jnp.max(scores, axis=1, keepdims=True)
            w = jnp.exp(scores - m)
            s = jnp.sum(w, axis=1, keepdims=True)
            w = (w / s).astype(jnp.bfloat16)
            ctx_h = jax.lax.dot(w, v_h, preferred_element_type=jnp.float32)
            ctx_ref[:, h * DH:(h + 1) * DH] = ctx_h.astype(jnp.bfloat16)

        wob = wo_ref[:, :].astype(jnp.bfloat16)
        partial = jax.lax.dot(ctx_ref[:, :], wob,
                              preferred_element_type=jnp.float32)
        for c in range(N_DEV):
            acc_ref[c] = partial[c * CHUNK:(c + 1) * CHUNK, :]

        for s in range(N_DEV - 1):
            c_send = lax.rem(my_pos - s + 2 * N_DEV, N_DEV)
            c_recv = lax.rem(my_pos - s - 1 + 2 * N_DEV, N_DEV)
            stage_ref[:, :] = acc_ref[c_send].astype(jnp.bfloat16)
            rdma = pltpu.make_async_remote_copy(
                src_ref=stage_ref,
                dst_ref=rs_recv_ref.at[s],
                send_sem=rs_send_sems.at[s],
                recv_sem=rs_recv_sems.at[s],
                device_id=(right,),
                device_id_type=pl.DeviceIdType.MESH,
            )
            rdma.start()
            rdma.wait()
            acc_ref[c_recv] = acc_ref[c_recv] + rs_recv_ref[s].astype(jnp.float32)

        c_own = lax.rem(my_pos + 1, N_DEV)
        out_ref[0, pl.ds(c_own * CHUNK, CHUNK), :] = acc_ref[c_own]

        stage_ref[:, :] = acc_ref[c_own].astype(jnp.bfloat16)
        for t in range(N_DEV - 1):
            src = stage_ref if t == 0 else ag_recv_ref.at[t - 1]
            rdma = pltpu.make_async_remote_copy(
                src_ref=src,
                dst_ref=ag_recv_ref.at[t],
                send_sem=ag_send_sems.at[t],
                recv_sem=ag_recv_sems.at[t],
                device_id=(right,),
                device_id_type=pl.DeviceIdType.MESH,
            )
            rdma.start()
            rdma.wait()
            c_r = lax.rem(my_pos - t + 2 * N_DEV, N_DEV)
            out_ref[0, pl.ds(c_r * CHUNK, CHUNK), :] = (
                ag_recv_ref[t].astype(jnp.float32))

    return pl.pallas_call(
        body,
        out_shape=jax.ShapeDtypeStruct((1, SQ, D_MODEL), jnp.float32),
        in_specs=[pl.BlockSpec(memory_space=pltpu.VMEM)] * 5,
        out_specs=pl.BlockSpec(memory_space=pltpu.VMEM),
        scratch_shapes=[
            pltpu.VMEM((N_DEV, CHUNK, D_MODEL), jnp.float32),
            pltpu.VMEM((SQ, H_PER * DH), jnp.bfloat16),
            pltpu.VMEM((CHUNK, D_MODEL), jnp.bfloat16),
            pltpu.VMEM((N_DEV - 1, CHUNK, D_MODEL), jnp.bfloat16),
            pltpu.VMEM((N_DEV - 1, CHUNK, D_MODEL), jnp.bfloat16),
            pltpu.SemaphoreType.DMA((N_DEV - 1,)),
            pltpu.SemaphoreType.DMA((N_DEV - 1,)),
            pltpu.SemaphoreType.DMA((N_DEV - 1,)),
            pltpu.SemaphoreType.DMA((N_DEV - 1,)),
        ],
        compiler_params=pltpu.CompilerParams(collective_id=0),
    )(x2, wq_my, k3, v3, wo_my)


# baseline (device time: 110604 ns/iter reference)
import jax
import jax.numpy as jnp
from jax import lax
from jax.experimental import pallas as pl
from jax.experimental.pallas import tpu as pltpu

N_DEV = 8
SQ = 1024
SKV = 1024
H_PER = 8
DH = 128
D_MODEL = 1024
CHUNK = SQ // N_DEV
SCALE = 0.08838834764831843


def kernel(x, Wq, K_ext, V_ext, Wo):
    my = lax.axis_index("i")
    wq_my = lax.dynamic_slice(Wq, (0, my * H_PER * DH), (D_MODEL, H_PER * DH))
    wo_my = lax.dynamic_slice(Wo, (my * H_PER * DH, 0), (H_PER * DH, D_MODEL))
    x2 = x.reshape(SQ, D_MODEL)
    k3 = K_ext.reshape(SKV, H_PER, DH)
    v3 = V_ext.reshape(SKV, H_PER, DH)

    def body(x_ref, wq_ref, k_ref, v_ref, wo_ref, out_ref,
             acc_ref, ctx_ref, stage_ref, rs_recv_ref, ag_recv_ref,
             rs_send_sems, rs_recv_sems, ag_send_sems, ag_recv_sems):
        my_pos = lax.axis_index("i")
        right = lax.rem(my_pos + 1, N_DEV)

        xb = x_ref[:, :].astype(jnp.bfloat16)
        wqb = wq_ref[:, :].astype(jnp.bfloat16)
        q_all = jax.lax.dot(xb, wqb, preferred_element_type=jnp.float32)
        q_all = q_all.astype(jnp.bfloat16)

        qi = lax.broadcasted_iota(jnp.int32, (SQ, SKV), 0)
        ki = lax.broadcasted_iota(jnp.int32, (SQ, SKV), 1)
        mask = (jnp.abs(qi - ki) <= 128) | (ki < 32) | (qi < 32)

        for h in range(H_PER):
            q_h = q_all[:, h * DH:(h + 1) * DH]
            k_h = k_ref[:, h, :].astype(jnp.bfloat16)
            v_h = v_ref[:, h, :].astype(jnp.bfloat16)
            scores = lax.dot_general(
                q_h, k_h,
                dimension_numbers=(((1,), (1,)), ((), ())),
                preferred_element_type=jnp.float32,
            ) * SCALE
            scores = jnp.where(mask, scores, -1e9)
            m = jnp.max(scores, axis=1, keepdims=True)
            w = jnp.exp(scores - m)
            s = jnp.sum(w, axis=1, keepdims=True)
            w = (w / s).astype(jnp.bfloat16)
            ctx_h = jax.lax.dot(w, v_h, preferred_element_type=jnp.float32)
            ctx_ref[:, h * DH:(h + 1) * DH] = ctx_h.astype(jnp.bfloat16)

        wob = wo_ref[:, :].astype(jnp.bfloat16)
        partial = jax.lax.dot(ctx_ref[:, :], wob,
                              preferred_element_type=jnp.float32)
        for c in range(N_DEV):
            acc_ref[c] = partial[c * CHUNK:(c + 1) * CHUNK, :]

        for s in range(N_DEV - 1):
            c_send = lax.rem(my_pos - s + 2 * N_DEV, N_DEV)
            c_recv = lax.rem(my_pos - s - 1 + 2 * N_DEV, N_DEV)
            stage_ref[:, :] = acc_ref[c_send].astype(jnp.bfloat16)
            rdma = pltpu.make_async_remote_copy(
                src_ref=stage_ref,
                dst_ref=rs_recv_ref.at[s],
                send_sem=rs_send_sems.at[s],
                recv_sem=rs_recv_sems.at[s],
                device_id=(right,),
                device_id_type=pl.DeviceIdType.MESH,
            )
            rdma.start()
            rdma.wait()
            acc_ref[c_recv] = acc_ref[c_recv] + rs_recv_ref[s].astype(jnp.float32)

        c_own = lax.rem(my_pos + 1, N_DEV)
        out_ref[0, pl.ds(c_own * CHUNK, CHUNK), :] = acc_ref[c_own]

        stage_ref[:, :] = acc_ref[c_own].astype(jnp.bfloat16)
        for t in range(N_DEV - 1):
            src = stage_ref if t == 0 else ag_recv_ref.at[t - 1]
            rdma = pltpu.make_async_remote_copy(
                src_ref=src,
                dst_ref=ag_recv_ref.at[t],
                send_sem=ag_send_sems.at[t],
                recv_sem=ag_recv_sems.at[t],
                device_id=(right,),
                device_id_type=pl.DeviceIdType.MESH,
            )
            rdma.start()
            rdma.wait()
            c_r = lax.rem(my_pos - t + 2 * N_DEV, N_DEV)
            out_ref[0, pl.ds(c_r * CHUNK, CHUNK), :] = (
                ag_recv_ref[t].astype(jnp.float32))

    return pl.pallas_call(
        body,
        out_shape=jax.ShapeDtypeStruct((1, SQ, D_MODEL), jnp.float32),
        in_specs=[pl.BlockSpec(memory_space=pltpu.VMEM)] * 5,
        out_specs=pl.BlockSpec(memory_space=pltpu.VMEM),
        scratch_shapes=[
            pltpu.VMEM((N_DEV, CHUNK, D_MODEL), jnp.float32),
            pltpu.VMEM((SQ, H_PER * DH), jnp.bfloat16),
            pltpu.VMEM((CHUNK, D_MODEL), jnp.bfloat16),
            pltpu.VMEM((N_DEV - 1, CHUNK, D_MODEL), jnp.bfloat16),
            pltpu.VMEM((N_DEV - 1, CHUNK, D_MODEL), jnp.bfloat16),
            pltpu.SemaphoreType.DMA((N_DEV - 1,)),
            pltpu.SemaphoreType.DMA((N_DEV - 1,)),
            pltpu.SemaphoreType.DMA((N_DEV - 1,)),
            pltpu.SemaphoreType.DMA((N_DEV - 1,)),
        ],
    )(x2, wq_my, k3, v3, wo_my)


# device time: 94972 ns/iter; 1.1646x vs baseline; 1.1646x over previous
import jax
import jax.numpy as jnp
from jax import lax
from jax.experimental import pallas as pl
from jax.experimental.pallas import tpu as pltpu

N_DEV = 8
SQ = 1024
SKV = 1024
H_PER = 8
DH = 128
D_MODEL = 1024
CHUNK = SQ // N_DEV
SCALE = 0.08838834764831843


def kernel(x, Wq, K_ext, V_ext, Wo):
    my = lax.axis_index("i")
    wq_my = lax.dynamic_slice(Wq, (0, my * H_PER * DH), (D_MODEL, H_PER * DH))
    wo_my = lax.dynamic_slice(Wo, (my * H_PER * DH, 0), (H_PER * DH, D_MODEL))
    x2 = x.reshape(SQ, D_MODEL)
    k3 = K_ext.reshape(SKV, H_PER, DH)
    v3 = V_ext.reshape(SKV, H_PER, DH)

    def body(x_ref, wq_ref, k_ref, v_ref, wo_ref, out_ref,
             acc_ref, ctx_ref, gath_ref,
             stage0_ref, stage1_ref, stage2_ref,
             rs_recv0_ref, rs_recv1_ref, rs_recv2_ref,
             rs_send_sems, rs_recv_sems, ag_send_sems, ag_recv_sems):
        my_pos = lax.axis_index("i")

        xb = x_ref[:, :].astype(jnp.bfloat16)
        wqb = wq_ref[:, :].astype(jnp.bfloat16)
        q_all = jax.lax.dot(xb, wqb, preferred_element_type=jnp.float32)
        q_all = q_all.astype(jnp.bfloat16)

        qi = lax.broadcasted_iota(jnp.int32, (SQ, SKV), 0)
        ki = lax.broadcasted_iota(jnp.int32, (SQ, SKV), 1)
        mask = (jnp.abs(qi - ki) <= 128) | (ki < 32) | (qi < 32)

        for h in range(H_PER):
            q_h = q_all[:, h * DH:(h + 1) * DH]
            k_h = k_ref[:, h, :].astype(jnp.bfloat16)
            v_h = v_ref[:, h, :].astype(jnp.bfloat16)
            scores = lax.dot_general(
                q_h, k_h,
                dimension_numbers=(((1,), (1,)), ((), ())),
                preferred_element_type=jnp.float32,
            ) * SCALE
            scores = jnp.where(mask, scores, -1e9)
            m = jnp.max(scores, axis=1, keepdims=True)
            w = jnp.exp(scores - m)
            s = jnp.sum(w, axis=1, keepdims=True)
            w = (w / s).astype(jnp.bfloat16)
            ctx_h = jax.lax.dot(w, v_h, preferred_element_type=jnp.float32)
            ctx_ref[:, h * DH:(h + 1) * DH] = ctx_h.astype(jnp.bfloat16)

        wob = wo_ref[:, :].astype(jnp.bfloat16)
        partial = jax.lax.dot(ctx_ref[:, :], wob,
                              preferred_element_type=jnp.float32)
        acc_ref[:, :] = partial

        rs_stages = [stage0_ref, stage1_ref, stage2_ref]
        rs_recvs = [rs_recv0_ref, rs_recv1_ref, rs_recv2_ref]
        base = my_pos * 0
        for r, (m, sel) in enumerate([(4, 2), (3, 1), (1, 0)]):
            L = 512 >> r
            b = (my_pos >> sel) & 1
            partner = my_pos ^ m
            send_off = base + (1 - b) * L
            stg = rs_stages[r]
            stg[:, :] = acc_ref[pl.ds(send_off, L), :].astype(jnp.bfloat16)
            rdma = pltpu.make_async_remote_copy(
                src_ref=stg,
                dst_ref=rs_recvs[r],
                send_sem=rs_send_sems.at[r],
                recv_sem=rs_recv_sems.at[r],
                device_id=(partner,),
                device_id_type=pl.DeviceIdType.MESH,
            )
            rdma.start()
            rdma.wait()
            base = base + b * L
            acc_ref[pl.ds(base, L), :] = (
                acc_ref[pl.ds(base, L), :]
                + rs_recvs[r][:, :].astype(jnp.float32))

        out_ref[0, pl.ds(base, CHUNK), :] = acc_ref[pl.ds(base, CHUNK), :]
        gath_ref[pl.ds(base, CHUNK), :] = (
            acc_ref[pl.ds(base, CHUNK), :].astype(jnp.bfloat16))

        for t, m in enumerate([1, 3, 4]):
            L = CHUNK << t
            partner = my_pos ^ m
            sbase = (my_pos & ~((1 << t) - 1)) * CHUNK
            pbase = (partner & ~((1 << t) - 1)) * CHUNK
            rdma = pltpu.make_async_remote_copy(
                src_ref=gath_ref.at[pl.ds(sbase, L)],
                dst_ref=gath_ref.at[pl.ds(sbase, L)],
                send_sem=ag_send_sems.at[t],
                recv_sem=ag_recv_sems.at[t],
                device_id=(partner,),
                device_id_type=pl.DeviceIdType.MESH,
            )
            rdma.start()
            rdma.wait()
            out_ref[0, pl.ds(pbase, L), :] = (
                gath_ref[pl.ds(pbase, L), :].astype(jnp.float32))

    return pl.pallas_call(
        body,
        out_shape=jax.ShapeDtypeStruct((1, SQ, D_MODEL), jnp.float32),
        in_specs=[pl.BlockSpec(memory_space=pltpu.VMEM)] * 5,
        out_specs=pl.BlockSpec(memory_space=pltpu.VMEM),
        scratch_shapes=[
            pltpu.VMEM((SQ, D_MODEL), jnp.float32),
            pltpu.VMEM((SQ, H_PER * DH), jnp.bfloat16),
            pltpu.VMEM((SQ, D_MODEL), jnp.bfloat16),
            pltpu.VMEM((512, D_MODEL), jnp.bfloat16),
            pltpu.VMEM((256, D_MODEL), jnp.bfloat16),
            pltpu.VMEM((128, D_MODEL), jnp.bfloat16),
            pltpu.VMEM((512, D_MODEL), jnp.bfloat16),
            pltpu.VMEM((256, D_MODEL), jnp.bfloat16),
            pltpu.VMEM((128, D_MODEL), jnp.bfloat16),
            pltpu.SemaphoreType.DMA((3,)),
            pltpu.SemaphoreType.DMA((3,)),
            pltpu.SemaphoreType.DMA((3,)),
            pltpu.SemaphoreType.DMA((3,)),
        ],
    )(x2, wq_my, k3, v3, wo_my)


# device time: 76295 ns/iter; 1.4497x vs baseline; 1.2448x over previous
import jax
import jax.numpy as jnp
from jax import lax
from jax.experimental import pallas as pl
from jax.experimental.pallas import tpu as pltpu

N_DEV = 8
SQ = 1024
SKV = 1024
H_PER = 8
DH = 128
D_MODEL = 1024
CHUNK = SQ // N_DEV
SCALE = 0.08838834764831843


def kernel(x, Wq, K_ext, V_ext, Wo):
    my = lax.axis_index("i")
    wq_my = lax.dynamic_slice(Wq, (0, my * H_PER * DH), (D_MODEL, H_PER * DH))
    wo_my = lax.dynamic_slice(Wo, (my * H_PER * DH, 0), (H_PER * DH, D_MODEL))
    x2 = x.reshape(SQ, D_MODEL)
    k3 = K_ext.reshape(SKV, H_PER, DH)
    v3 = V_ext.reshape(SKV, H_PER, DH)

    def body(x_ref, wq_ref, k_ref, v_ref, wo_ref, out_ref,
             acc_ref, ctx_ref, gath_ref,
             st00, st10, st20, st01, st11, st21,
             rv00, rv10, rv20, rv01, rv11, rv21,
             rs_send_sems, rs_recv_sems, ag_send_sems, ag_recv_sems):
        my_pos = lax.axis_index("i")

        xb = x_ref[:, :].astype(jnp.bfloat16)
        wqb = wq_ref[:, :].astype(jnp.bfloat16)
        q_all = jax.lax.dot(xb, wqb, preferred_element_type=jnp.float32)
        q_all = q_all.astype(jnp.bfloat16)

        qi = lax.broadcasted_iota(jnp.int32, (SQ, SKV), 0)
        ki = lax.broadcasted_iota(jnp.int32, (SQ, SKV), 1)
        mask = (jnp.abs(qi - ki) <= 128) | (ki < 32) | (qi < 32)

        for h in range(H_PER):
            q_h = q_all[:, h * DH:(h + 1) * DH]
            k_h = k_ref[:, h, :].astype(jnp.bfloat16)
            v_h = v_ref[:, h, :].astype(jnp.bfloat16)
            scores = lax.dot_general(
                q_h, k_h,
                dimension_numbers=(((1,), (1,)), ((), ())),
                preferred_element_type=jnp.float32,
            ) * SCALE
            scores = jnp.where(mask, scores, -1e9)
            m = jnp.max(scores, axis=1, keepdims=True)
            w = jnp.exp(scores - m)
            s = jnp.sum(w, axis=1, keepdims=True)
            w = (w / s).astype(jnp.bfloat16)
            ctx_h = jax.lax.dot(w, v_h, preferred_element_type=jnp.float32)
            ctx_ref[:, h * DH:(h + 1) * DH] = ctx_h.astype(jnp.bfloat16)

        wob = wo_ref[:, :].astype(jnp.bfloat16)
        partial = jax.lax.dot(ctx_ref[:, :], wob,
                              preferred_element_type=jnp.float32)
        acc_ref[:, :] = partial

        b0 = my_pos & 1
        b1 = (my_pos >> 1) & 1
        b2 = (my_pos >> 2) & 1
        RS_CFG = [[(4, 2), (3, 1), (1, 0)], [(3, 1), (1, 0), (4, 2)]]
        AG_MASKS = [[1, 3, 4], [4, 1, 3]]
        COLS = [(0, 512), (512, 1024)]
        stages = [[st00, st01], [st10, st11], [st20, st21]]
        recvs = [[rv00, rv01], [rv10, rv11], [rv20, rv21]]

        base = [my_pos * 0, my_pos * 0]
        for r in range(3):
            L = 512 >> r
            rdmas = []
            for p in range(2):
                m, sel = RS_CFG[p][r]
                c0, c1 = COLS[p]
                b = (my_pos >> sel) & 1
                partner = my_pos ^ m
                send_off = base[p] + (1 - b) * L
                stg = stages[r][p]
                stg[:, :] = acc_ref[pl.ds(send_off, L), c0:c1].astype(
                    jnp.bfloat16)
                rdma = pltpu.make_async_remote_copy(
                    src_ref=stg,
                    dst_ref=recvs[r][p],
                    send_sem=rs_send_sems.at[r, p],
                    recv_sem=rs_recv_sems.at[r, p],
                    device_id=(partner,),
                    device_id_type=pl.DeviceIdType.MESH,
                )
                rdma.start()
                rdmas.append(rdma)
                base[p] = base[p] + b * L
            for p in range(2):
                rdmas[p].wait()
            for p in range(2):
                c0, c1 = COLS[p]
                acc_ref[pl.ds(base[p], L), c0:c1] = (
                    acc_ref[pl.ds(base[p], L), c0:c1]
                    + recvs[r][p][:, :].astype(jnp.float32))

        jown = [my_pos, 4 * b1 + 2 * b0 + b2]
        for p in range(2):
            c0, c1 = COLS[p]
            o = base[p]
            out_ref[0, pl.ds(o, CHUNK), c0:c1] = acc_ref[pl.ds(o, CHUNK), c0:c1]
            gath_ref[pl.ds(o, CHUNK), c0:c1] = (
                acc_ref[pl.ds(o, CHUNK), c0:c1].astype(jnp.bfloat16))

        for t in range(3):
            L = CHUNK << t
            rdmas = []
            pbases = []
            for p in range(2):
                m = AG_MASKS[p][t]
                c0, c1 = COLS[p]
                partner = my_pos ^ m
                pb0 = partner & 1
                pb1 = (partner >> 1) & 1
                pb2 = (partner >> 2) & 1
                jp = partner if p == 0 else 4 * pb1 + 2 * pb0 + pb2
                sbase = (jown[p] & ~((1 << t) - 1)) * CHUNK
                pbase = (jp & ~((1 << t) - 1)) * CHUNK
                pbases.append(pbase)
                rdma = pltpu.make_async_remote_copy(
                    src_ref=gath_ref.at[pl.ds(sbase, L), pl.ds(c0, 512)],
                    dst_ref=gath_ref.at[pl.ds(sbase, L), pl.ds(c0, 512)],
                    send_sem=ag_send_sems.at[t, p],
                    recv_sem=ag_recv_sems.at[t, p],
                    device_id=(partner,),
                    device_id_type=pl.DeviceIdType.MESH,
                )
                rdma.start()
                rdmas.append(rdma)
            for p in range(2):
                rdmas[p].wait()
            for p in range(2):
                c0, c1 = COLS[p]
                out_ref[0, pl.ds(pbases[p], L), c0:c1] = (
                    gath_ref[pl.ds(pbases[p], L), c0:c1].astype(jnp.float32))

    return pl.pallas_call(
        body,
        out_shape=jax.ShapeDtypeStruct((1, SQ, D_MODEL), jnp.float32),
        in_specs=[pl.BlockSpec(memory_space=pltpu.VMEM)] * 5,
        out_specs=pl.BlockSpec(memory_space=pltpu.VMEM),
        scratch_shapes=[
            pltpu.VMEM((SQ, D_MODEL), jnp.float32),
            pltpu.VMEM((SQ, H_PER * DH), jnp.bfloat16),
            pltpu.VMEM((SQ, D_MODEL), jnp.bfloat16),
            pltpu.VMEM((512, 512), jnp.bfloat16),
            pltpu.VMEM((256, 512), jnp.bfloat16),
            pltpu.VMEM((128, 512), jnp.bfloat16),
            pltpu.VMEM((512, 512), jnp.bfloat16),
            pltpu.VMEM((256, 512), jnp.bfloat16),
            pltpu.VMEM((128, 512), jnp.bfloat16),
            pltpu.VMEM((512, 512), jnp.bfloat16),
            pltpu.VMEM((256, 512), jnp.bfloat16),
            pltpu.VMEM((128, 512), jnp.bfloat16),
            pltpu.VMEM((512, 512), jnp.bfloat16),
            pltpu.VMEM((256, 512), jnp.bfloat16),
            pltpu.VMEM((128, 512), jnp.bfloat16),
            pltpu.SemaphoreType.DMA((3, 2)),
            pltpu.SemaphoreType.DMA((3, 2)),
            pltpu.SemaphoreType.DMA((3, 2)),
            pltpu.SemaphoreType.DMA((3, 2)),
        ],
    )(x2, wq_my, k3, v3, wo_my)
